# baseline (speedup 1.0000x reference)
import jax
import jax.numpy as jnp
from jax import lax
from jax.experimental import pallas as pl
from jax.experimental.pallas import tpu as pltpu
from jax.experimental.pallas import tpu_sc as plsc

B, H, W = 4, 224, 224
C = 128
NROW = 224
L = 16
NPOS = H * W
NPB = 8
P = NPOS // NPB
N = 112
NCH = P // N


def _body(i_hbm, j_hbm, ti_hbm, tj_hbm, out_hbm,
          ti_v, tj_v, idx_i, idx_j, outbuf, sem0, sem1):
    info = plsc.get_sparse_core_info()
    nc, ns = info.num_cores, info.num_subcores

    wid = lax.axis_index("s") * nc + lax.axis_index("c")
    b = wid // NPB
    p0 = (wid % NPB) * P

    pltpu.sync_copy(ti_hbm, ti_v)
    pltpu.sync_copy(tj_hbm, tj_v)
    pltpu.sync_copy(i_hbm.at[pl.ds(b * NPOS + p0, P)], idx_i)
    pltpu.sync_copy(j_hbm.at[pl.ds(b * NPOS + p0, P)], idx_j)

    sems = (sem0, sem1)
    lane = lax.broadcasted_iota(jnp.int32, (L,), 0)
    cvecs = [cb * L + lane for cb in range(C // L)]

    def fill_slab(ck, buf):
        @plsc.parallel_loop(0, N, 1, unroll=4)
        def pos_loop(p):
            pv = lane * 0 + (ck * N + p)
            bi = plsc.load_gather(idx_i, [pv]) * C
            bj = plsc.load_gather(idx_j, [pv]) * C
            for cb in range(C // L):
                v = plsc.load_gather(ti_v, [bi + cvecs[cb]])
                outbuf[buf, p, pl.ds(cb * L, L)] = v
                w = plsc.load_gather(tj_v, [bj + cvecs[cb]])
                outbuf[buf, p, pl.ds(C + cb * L, L)] = w

    def out_dma(ck, buf, sem):
        return pltpu.make_async_copy(
            outbuf.at[buf],
            out_hbm.at[b, pl.ds(p0 + ck * N, N), :],
            sem,
        )

    def ck_pair(s, _):
        for k in range(2):
            ck = 2 * s + k

            @pl.when(s >= 1)
            def _():
                out_dma(0, k, sems[k]).wait()

            fill_slab(ck, k)
            out_dma(ck, k, sems[k]).start()
        return 0

    lax.fori_loop(0, NCH // 2, ck_pair, 0)

    out_dma(0, 0, sem0).wait()
    out_dma(0, 1, sem1).wait()


@jax.jit
def _position_embedding_sc(i, j, table_i, table_j):
    mesh = plsc.VectorSubcoreMesh(core_axis_name="c", subcore_axis_name="s")
    fn = pl.kernel(
        _body,
        out_type=jax.ShapeDtypeStruct((B, NPOS, 2 * C), jnp.float32),
        mesh=mesh,
        scratch_types=[
            pltpu.VMEM((NROW * C,), jnp.float32),
            pltpu.VMEM((NROW * C,), jnp.float32),
            pltpu.VMEM((P,), jnp.int32),
            pltpu.VMEM((P,), jnp.int32),
            pltpu.VMEM((2, N, 2 * C), jnp.float32),
            pltpu.SemaphoreType.DMA,
            pltpu.SemaphoreType.DMA,
        ],
        compiler_params=pltpu.CompilerParams(needs_layout_passes=False),
    )
    outp = fn(i.reshape(-1), j.reshape(-1),
              table_i.reshape(-1), table_j.reshape(-1))
    return jnp.transpose(outp.reshape(B, H, W, 2 * C), (0, 3, 1, 2))


def kernel(i, j, table_i, table_j):
    return _position_embedding_sc(i, j, table_i, table_j)


# speedup vs baseline: 1.0699x; 1.0699x over previous
import jax
import jax.numpy as jnp
from jax import lax
from jax.experimental import pallas as pl
from jax.experimental.pallas import tpu as pltpu
from jax.experimental.pallas import tpu_sc as plsc

B, H, W = 4, 224, 224
C = 128
NROW = 224
L = 16
NPOS = H * W
NPB = 8
P = NPOS // NPB
N = 64
NCH = P // N


def _body(i_hbm, j_hbm, ti_hbm, tj_hbm, out_hbm,
          ti_v, tj_v, idx_i, idx_j, outbuf, sem0, sem1):
    info = plsc.get_sparse_core_info()
    nc, ns = info.num_cores, info.num_subcores

    wid = lax.axis_index("s") * nc + lax.axis_index("c")
    b = wid // NPB
    p0 = (wid % NPB) * P

    pltpu.sync_copy(ti_hbm, ti_v)
    pltpu.sync_copy(tj_hbm, tj_v)
    pltpu.sync_copy(i_hbm.at[pl.ds(b * NPOS + p0, P)], idx_i)
    pltpu.sync_copy(j_hbm.at[pl.ds(b * NPOS + p0, P)], idx_j)

    sems = (sem0, sem1)
    lane = lax.broadcasted_iota(jnp.int32, (L,), 0)
    cvecs = [cb * L + lane for cb in range(C // L)]

    def fill_slab(ck, buf):
        @plsc.parallel_loop(0, N, 1, unroll=4)
        def pos_loop(p):
            pv = lane * 0 + (ck * N + p)
            bi = plsc.load_gather(idx_i, [pv]) * C
            bj = plsc.load_gather(idx_j, [pv]) * C
            for cb in range(C // L):
                v = plsc.load_gather(ti_v, [bi + cvecs[cb]])
                outbuf[buf, p, pl.ds(cb * L, L)] = v
                w = plsc.load_gather(tj_v, [bj + cvecs[cb]])
                outbuf[buf, p, pl.ds(C + cb * L, L)] = w

    def out_dma(ck, buf, sem):
        return pltpu.make_async_copy(
            outbuf.at[buf],
            out_hbm.at[b, pl.ds(p0 + ck * N, N), :],
            sem,
        )

    def ck_pair(s, _):
        for k in range(2):
            ck = 2 * s + k

            @pl.when(s >= 1)
            def _():
                out_dma(0, k, sems[k]).wait()

            fill_slab(ck, k)
            out_dma(ck, k, sems[k]).start()
        return 0

    lax.fori_loop(0, NCH // 2, ck_pair, 0)

    out_dma(0, 0, sem0).wait()
    out_dma(0, 1, sem1).wait()


@jax.jit
def _position_embedding_sc(i, j, table_i, table_j):
    mesh = plsc.VectorSubcoreMesh(core_axis_name="c", subcore_axis_name="s")
    fn = pl.kernel(
        _body,
        out_type=jax.ShapeDtypeStruct((B, NPOS, 2 * C), jnp.float32),
        mesh=mesh,
        scratch_types=[
            pltpu.VMEM((NROW * C,), jnp.float32),
            pltpu.VMEM((NROW * C,), jnp.float32),
            pltpu.VMEM((P,), jnp.int32),
            pltpu.VMEM((P,), jnp.int32),
            pltpu.VMEM((2, N, 2 * C), jnp.float32),
            pltpu.SemaphoreType.DMA,
            pltpu.SemaphoreType.DMA,
        ],
        compiler_params=pltpu.CompilerParams(needs_layout_passes=False),
    )
    outp = fn(i.reshape(-1), j.reshape(-1),
              table_i.reshape(-1), table_j.reshape(-1))
    return jnp.transpose(outp.reshape(B, H, W, 2 * C), (0, 3, 1, 2))


def kernel(i, j, table_i, table_j):
    return _position_embedding_sc(i, j, table_i, table_j)
